# baseline (speedup 1.0000x reference)
import jax
import jax.numpy as jnp
from jax import lax
from jax.experimental import pallas as pl
from jax.experimental.pallas import tpu as pltpu
from jax.experimental.pallas import tpu_sc as plsc

N = 10000
D = 128
E = 320000
NC = 2
NS = 16
NW = NC * NS
EDGES_PER_TILE = E // NW
CHUNK = 128
NCH = 80
NCH2 = NCH // 2
PAD = NCH * CHUNK - EDGES_PER_TILE
ROWS_PER_TILE = 624
ROWS_TAIL = N - NS * ROWS_PER_TILE
SUP_ROWS = N + 8


def _mm_body(x_ref, w_ref, o_ref):
    o_ref[pl.ds(0, N), :] = jnp.dot(x_ref[...], w_ref[...],
                                    preferred_element_type=jnp.float32)
    o_ref[pl.ds(N, SUP_ROWS - N), :] = jnp.zeros((SUP_ROWS - N, D),
                                                 jnp.float32)


def _matmul(x, w):
    return pl.pallas_call(
        _mm_body,
        out_shape=jax.ShapeDtypeStruct((SUP_ROWS, w.shape[1]), jnp.float32),
    )(x, w)


def _merge_mm_body(p_ref, b_ref, w_ref, o_ref):
    h = p_ref[0] + p_ref[1] + b_ref[...]
    h = jnp.where(h >= 0, h, 0.25 * h)
    o_ref[pl.ds(0, N), :] = jnp.dot(h, w_ref[...],
                                    preferred_element_type=jnp.float32)
    o_ref[pl.ds(N, SUP_ROWS - N), :] = jnp.zeros((SUP_ROWS - N, D),
                                                 jnp.float32)


def _merge_matmul(partials, b, w):
    return pl.pallas_call(
        _merge_mm_body,
        out_shape=jax.ShapeDtypeStruct((SUP_ROWS, w.shape[1]), jnp.float32),
    )(partials, b, w)


def _merge_act_body(p_ref, b_ref, o_ref):
    h = p_ref[0] + p_ref[1] + b_ref[...]
    o_ref[...] = jnp.where(h >= 0, h, 0.25 * h)


def _merge_act(partials, b):
    return pl.pallas_call(
        _merge_act_body,
        out_shape=jax.ShapeDtypeStruct((N, D), jnp.float32),
    )(partials, b)


def _sc_scatter_body(sup_hbm, rowp_hbm, colp_hbm, zero_hbm, out_hbm,
                     colv, rowv, gat0, gat1, gsem0, gsem1, acc):
    cid = lax.axis_index("c")
    sid = lax.axis_index("s")
    wid = cid * NS + sid
    rbase = sid * ROWS_PER_TILE

    pltpu.sync_copy(zero_hbm.at[pl.ds(rbase, ROWS_PER_TILE)],
                    acc.at[pl.ds(rbase, ROWS_PER_TILE)])

    @pl.when(sid == NS - 1)
    def _():
        pltpu.sync_copy(zero_hbm.at[pl.ds(NS * ROWS_PER_TILE, ROWS_TAIL)],
                        acc.at[pl.ds(NS * ROWS_PER_TILE, ROWS_TAIL)])

    plsc.subcore_barrier()

    for h in range(2):
        pltpu.sync_copy(colp_hbm.at[wid, pl.ds(h * NCH2, NCH2)], colv)
        pltpu.sync_copy(rowp_hbm.at[wid, pl.ds(h * NCH2, NCH2)], rowv)

        @pl.loop(0, NCH2 // 2)
        def _(m):
            j = 2 * m
            a = pltpu.async_copy(sup_hbm.at[colv.at[j]], gat0, gsem0)
            b = pltpu.async_copy(sup_hbm.at[colv.at[j + 1]], gat1, gsem1)
            a.wait()
            pltpu.sync_copy(gat0, acc.at[rowv.at[j]], add=True)
            b.wait()
            pltpu.sync_copy(gat1, acc.at[rowv.at[j + 1]], add=True)

    plsc.subcore_barrier()
    pltpu.sync_copy(acc.at[pl.ds(rbase, ROWS_PER_TILE)],
                    out_hbm.at[cid, pl.ds(rbase, ROWS_PER_TILE)])

    @pl.when(sid == NS - 1)
    def _():
        pltpu.sync_copy(acc.at[pl.ds(NS * ROWS_PER_TILE, ROWS_TAIL)],
                        out_hbm.at[cid, pl.ds(NS * ROWS_PER_TILE, ROWS_TAIL)])


def _sc_scatter_add(support, rowp, colp, zeros):
    mesh = plsc.VectorSubcoreMesh(core_axis_name="c", subcore_axis_name="s")
    k = pl.kernel(
        _sc_scatter_body,
        out_type=jax.ShapeDtypeStruct((NC, N, D), jnp.float32),
        mesh=mesh,
        scratch_types=[
            pltpu.VMEM((NCH2, CHUNK), jnp.int32),
            pltpu.VMEM((NCH2, CHUNK), jnp.int32),
            pltpu.VMEM((CHUNK, D), jnp.float32),
            pltpu.VMEM((CHUNK, D), jnp.float32),
            pltpu.SemaphoreType.DMA,
            pltpu.SemaphoreType.DMA,
            pltpu.VMEM_SHARED((N, D), jnp.float32),
        ],
    )
    return k(support, rowp, colp, zeros)


def kernel(x, edge_index, W1, b1, W2, b2):
    ei = edge_index.astype(jnp.int32)
    rowp = jnp.pad(ei[0].reshape(NW, EDGES_PER_TILE), ((0, 0), (0, PAD)),
                   constant_values=0).reshape(NW, NCH, CHUNK)
    colp = jnp.pad(ei[1].reshape(NW, EDGES_PER_TILE), ((0, 0), (0, PAD)),
                   constant_values=N).reshape(NW, NCH, CHUNK)
    zeros = jnp.zeros((N, D), jnp.float32)
    b1r = jnp.reshape(b1, (1, D))
    b2r = jnp.reshape(b2, (1, D))

    support1 = _matmul(x, W1)
    part1 = _sc_scatter_add(support1, rowp, colp, zeros)
    support2 = _merge_matmul(part1, b1r, W2)
    part2 = _sc_scatter_add(support2, rowp, colp, zeros)
    return _merge_act(part2, b2r)


# speedup vs baseline: 1.0378x; 1.0378x over previous
import jax
import jax.numpy as jnp
from jax import lax
from jax.experimental import pallas as pl
from jax.experimental.pallas import tpu as pltpu
from jax.experimental.pallas import tpu_sc as plsc

N = 10000
D = 128
E = 320000
NC = 2
NS = 16
NW = NC * NS
EDGES_PER_TILE = E // NW
CHUNK = 128
NCH = 80
NCH2 = NCH // 2
PAD = NCH * CHUNK - EDGES_PER_TILE
ROWS_PER_TILE = 624
ROWS_TAIL = N - NS * ROWS_PER_TILE
SUP_ROWS = N + 8


def _mm_body(x_ref, w_ref, o_ref):
    o_ref[pl.ds(0, N), :] = jnp.dot(x_ref[...], w_ref[...],
                                    preferred_element_type=jnp.float32)
    o_ref[pl.ds(N, SUP_ROWS - N), :] = jnp.zeros((SUP_ROWS - N, D),
                                                 jnp.float32)


def _matmul(x, w):
    return pl.pallas_call(
        _mm_body,
        out_shape=jax.ShapeDtypeStruct((SUP_ROWS, w.shape[1]), jnp.float32),
    )(x, w)


def _merge_mm_body(p_ref, b_ref, w_ref, o_ref):
    h = p_ref[0] + p_ref[1] + b_ref[...]
    h = jnp.where(h >= 0, h, 0.25 * h)
    o_ref[pl.ds(0, N), :] = jnp.dot(h, w_ref[...],
                                    preferred_element_type=jnp.float32)
    o_ref[pl.ds(N, SUP_ROWS - N), :] = jnp.zeros((SUP_ROWS - N, D),
                                                 jnp.float32)


def _merge_matmul(partials, b, w):
    return pl.pallas_call(
        _merge_mm_body,
        out_shape=jax.ShapeDtypeStruct((SUP_ROWS, w.shape[1]), jnp.float32),
    )(partials, b, w)


def _merge_act_body(p_ref, b_ref, o_ref):
    h = p_ref[0] + p_ref[1] + b_ref[...]
    o_ref[...] = jnp.where(h >= 0, h, 0.25 * h)


def _merge_act(partials, b):
    return pl.pallas_call(
        _merge_act_body,
        out_shape=jax.ShapeDtypeStruct((N, D), jnp.float32),
    )(partials, b)


def _sc_scatter_body(sup_hbm, rowp_hbm, colp_hbm, zero_hbm, out_hbm,
                     colv, rowv, gat0, gat1, gsem0, gsem1, ssem0, ssem1, acc):
    cid = lax.axis_index("c")
    sid = lax.axis_index("s")
    wid = cid * NS + sid
    rbase = sid * ROWS_PER_TILE
    gat = (gat0, gat1)
    gsem = (gsem0, gsem1)
    ssem = (ssem0, ssem1)

    pltpu.sync_copy(zero_hbm.at[pl.ds(rbase, ROWS_PER_TILE)],
                    acc.at[pl.ds(rbase, ROWS_PER_TILE)])

    @pl.when(sid == NS - 1)
    def _():
        pltpu.sync_copy(zero_hbm.at[pl.ds(NS * ROWS_PER_TILE, ROWS_TAIL)],
                        acc.at[pl.ds(NS * ROWS_PER_TILE, ROWS_TAIL)])

    plsc.subcore_barrier()

    for h in range(2):
        pltpu.sync_copy(colp_hbm.at[wid, pl.ds(h * NCH2, NCH2)], colv)
        pltpu.sync_copy(rowp_hbm.at[wid, pl.ds(h * NCH2, NCH2)], rowv)

        g = [None, None]
        s = [None, None]
        g[0] = pltpu.async_copy(sup_hbm.at[colv.at[0]], gat[0], gsem[0])
        for j in range(NCH2):
            cur = j % 2
            oth = 1 - cur
            g[cur].wait()
            if s[oth] is not None:
                s[oth].wait()
                s[oth] = None
            if j + 1 < NCH2:
                g[oth] = pltpu.async_copy(sup_hbm.at[colv.at[j + 1]],
                                          gat[oth], gsem[oth])
            s[cur] = pltpu.async_copy(gat[cur], acc.at[rowv.at[j]],
                                      ssem[cur], add=True)
        s[(NCH2 - 1) % 2].wait()

    plsc.subcore_barrier()
    pltpu.sync_copy(acc.at[pl.ds(rbase, ROWS_PER_TILE)],
                    out_hbm.at[cid, pl.ds(rbase, ROWS_PER_TILE)])

    @pl.when(sid == NS - 1)
    def _():
        pltpu.sync_copy(acc.at[pl.ds(NS * ROWS_PER_TILE, ROWS_TAIL)],
                        out_hbm.at[cid, pl.ds(NS * ROWS_PER_TILE, ROWS_TAIL)])


def _sc_scatter_add(support, rowp, colp, zeros):
    mesh = plsc.VectorSubcoreMesh(core_axis_name="c", subcore_axis_name="s")
    k = pl.kernel(
        _sc_scatter_body,
        out_type=jax.ShapeDtypeStruct((NC, N, D), jnp.float32),
        mesh=mesh,
        scratch_types=[
            pltpu.VMEM((NCH2, CHUNK), jnp.int32),
            pltpu.VMEM((NCH2, CHUNK), jnp.int32),
            pltpu.VMEM((CHUNK, D), jnp.float32),
            pltpu.VMEM((CHUNK, D), jnp.float32),
            pltpu.SemaphoreType.DMA,
            pltpu.SemaphoreType.DMA,
            pltpu.SemaphoreType.DMA,
            pltpu.SemaphoreType.DMA,
            pltpu.VMEM_SHARED((N, D), jnp.float32),
        ],
    )
    return k(support, rowp, colp, zeros)


def kernel(x, edge_index, W1, b1, W2, b2):
    ei = edge_index.astype(jnp.int32)
    rowp = jnp.pad(ei[0].reshape(NW, EDGES_PER_TILE), ((0, 0), (0, PAD)),
                   constant_values=0).reshape(NW, NCH, CHUNK)
    colp = jnp.pad(ei[1].reshape(NW, EDGES_PER_TILE), ((0, 0), (0, PAD)),
                   constant_values=N).reshape(NW, NCH, CHUNK)
    zeros = jnp.zeros((N, D), jnp.float32)
    b1r = jnp.reshape(b1, (1, D))
    b2r = jnp.reshape(b2, (1, D))

    support1 = _matmul(x, W1)
    part1 = _sc_scatter_add(support1, rowp, colp, zeros)
    support2 = _merge_matmul(part1, b1r, W2)
    part2 = _sc_scatter_add(support2, rowp, colp, zeros)
    return _merge_act(part2, b2r)


# speedup vs baseline: 1.1002x; 1.0601x over previous
import jax
import jax.numpy as jnp
from jax import lax
from jax.experimental import pallas as pl
from jax.experimental.pallas import tpu as pltpu
from jax.experimental.pallas import tpu_sc as plsc

N = 10000
D = 128
E = 320000
NC = 2
NS = 16
NW = NC * NS
EDGES_PER_TILE = E // NW
CHUNK = 128
NCH = 80
NCH2 = NCH // 2
PAD = NCH * CHUNK - EDGES_PER_TILE
ROWS_PER_TILE = 624
ROWS_TAIL = N - NS * ROWS_PER_TILE
SUP_ROWS = N + 8


def _mm_body(x_ref, w_ref, o_ref):
    o_ref[pl.ds(0, N), :] = jnp.dot(x_ref[...], w_ref[...],
                                    preferred_element_type=jnp.float32)
    o_ref[pl.ds(N, SUP_ROWS - N), :] = jnp.zeros((SUP_ROWS - N, D),
                                                 jnp.float32)


def _matmul(x, w):
    return pl.pallas_call(
        _mm_body,
        out_shape=jax.ShapeDtypeStruct((SUP_ROWS, w.shape[1]), jnp.float32),
    )(x, w)


def _merge_mm_body(p_ref, b_ref, w_ref, o_ref):
    h = p_ref[0] + p_ref[1] + b_ref[...]
    h = jnp.where(h >= 0, h, 0.25 * h)
    o_ref[pl.ds(0, N), :] = jnp.dot(h, w_ref[...],
                                    preferred_element_type=jnp.float32)
    o_ref[pl.ds(N, SUP_ROWS - N), :] = jnp.zeros((SUP_ROWS - N, D),
                                                 jnp.float32)


def _merge_matmul(partials, b, w):
    return pl.pallas_call(
        _merge_mm_body,
        out_shape=jax.ShapeDtypeStruct((SUP_ROWS, w.shape[1]), jnp.float32),
    )(partials, b, w)


def _merge_act_body(p_ref, b_ref, o_ref):
    h = p_ref[0] + p_ref[1] + b_ref[...]
    o_ref[...] = jnp.where(h >= 0, h, 0.25 * h)


def _merge_act(partials, b):
    return pl.pallas_call(
        _merge_act_body,
        out_shape=jax.ShapeDtypeStruct((N, D), jnp.float32),
    )(partials, b)


def _sc_scatter_body(sup_hbm, rowp_hbm, colp_hbm, zero_hbm, out_hbm,
                     *refs):
    colv = refs[0:3]
    rowv = refs[3:6]
    gat = refs[6:9]
    icsem = refs[9:12]
    irsem = refs[12:15]
    gsem = refs[15:18]
    ssem = refs[18:21]
    acc = refs[21]
    cid = lax.axis_index("c")
    sid = lax.axis_index("s")
    wid = cid * NS + sid
    rbase = sid * ROWS_PER_TILE

    pltpu.sync_copy(zero_hbm.at[pl.ds(rbase, ROWS_PER_TILE)],
                    acc.at[pl.ds(rbase, ROWS_PER_TILE)])

    @pl.when(sid == NS - 1)
    def _():
        pltpu.sync_copy(zero_hbm.at[pl.ds(NS * ROWS_PER_TILE, ROWS_TAIL)],
                        acc.at[pl.ds(NS * ROWS_PER_TILE, ROWS_TAIL)])

    plsc.subcore_barrier()

    ic = [None] * 3
    ir = [None] * 3
    g = [None] * 3
    s = [None] * 3

    def idx_issue(j):
        b = j % 3
        ic[b] = pltpu.async_copy(colp_hbm.at[wid, j], colv[b], icsem[b])
        ir[b] = pltpu.async_copy(rowp_hbm.at[wid, j], rowv[b], irsem[b])

    idx_issue(0)
    idx_issue(1)
    ic[0].wait()
    ir[0].wait()
    g[0] = pltpu.async_copy(sup_hbm.at[colv[0]], gat[0], gsem[0])
    for j in range(NCH):
        b = j % 3
        nb = (j + 2) % 3
        if s[nb] is not None:
            s[nb].wait()
            s[nb] = None
        if j + 2 < NCH:
            idx_issue(j + 2)
        if j + 1 < NCH:
            b1 = (j + 1) % 3
            ic[b1].wait()
            ir[b1].wait()
            g[b1] = pltpu.async_copy(sup_hbm.at[colv[b1]], gat[b1], gsem[b1])
        g[b].wait()
        s[b] = pltpu.async_copy(gat[b], acc.at[rowv[b]], ssem[b], add=True)
    for b in range(3):
        if s[b] is not None:
            s[b].wait()

    plsc.subcore_barrier()
    pltpu.sync_copy(acc.at[pl.ds(rbase, ROWS_PER_TILE)],
                    out_hbm.at[cid, pl.ds(rbase, ROWS_PER_TILE)])

    @pl.when(sid == NS - 1)
    def _():
        pltpu.sync_copy(acc.at[pl.ds(NS * ROWS_PER_TILE, ROWS_TAIL)],
                        out_hbm.at[cid, pl.ds(NS * ROWS_PER_TILE, ROWS_TAIL)])


def _sc_scatter_add(support, rowp, colp, zeros):
    mesh = plsc.VectorSubcoreMesh(core_axis_name="c", subcore_axis_name="s")
    k = pl.kernel(
        _sc_scatter_body,
        out_type=jax.ShapeDtypeStruct((NC, N, D), jnp.float32),
        mesh=mesh,
        scratch_types=(
            [pltpu.VMEM((CHUNK,), jnp.int32)] * 6
            + [pltpu.VMEM((CHUNK, D), jnp.float32)] * 3
            + [pltpu.SemaphoreType.DMA] * 12
            + [pltpu.VMEM_SHARED((N, D), jnp.float32)]
        ),
    )
    return k(support, rowp, colp, zeros)


def kernel(x, edge_index, W1, b1, W2, b2):
    ei = edge_index.astype(jnp.int32)
    rowp = jnp.pad(ei[0].reshape(NW, EDGES_PER_TILE), ((0, 0), (0, PAD)),
                   constant_values=0).reshape(NW, NCH, CHUNK)
    colp = jnp.pad(ei[1].reshape(NW, EDGES_PER_TILE), ((0, 0), (0, PAD)),
                   constant_values=N).reshape(NW, NCH, CHUNK)
    zeros = jnp.zeros((N, D), jnp.float32)
    b1r = jnp.reshape(b1, (1, D))
    b2r = jnp.reshape(b2, (1, D))

    support1 = _matmul(x, W1)
    part1 = _sc_scatter_add(support1, rowp, colp, zeros)
    support2 = _merge_matmul(part1, b1r, W2)
    part2 = _sc_scatter_add(support2, rowp, colp, zeros)
    return _merge_act(part2, b2r)
